# baseline (speedup 1.0000x reference)
import jax
import jax.numpy as jnp
from jax import lax
from jax.experimental import pallas as pl
from jax.experimental.pallas import tpu as pltpu
from jax.experimental.pallas import tpu_sc as plsc

_NC, _NS = 2, 16
_NW = _NC * _NS


def _splat16(x):
    return lax.broadcast(x, (16,))


def _make_sc_kernel(B, T, T_vis, D_vis, T_prop, D_prop, L, D_lang):
    D_out = D_vis + D_prop + D_lang
    SPW = B // _NW
    CH = 32
    NCH = T // CH
    LCH = 8
    inv_L = 1.0 / L

    def body(vis_f, vis_t, prop_f, prop_t, lang, tgt, out,
             tgt_v, vist_v, propt_v,
             visidxL_v, visidxR_v, visw_v,
             propidx_v, propw_v, propflat_v,
             langsum_v, langstage_v, fused_v,
             visL_v, visR_v, sem):
        wid = lax.axis_index("s") * _NC + lax.axis_index("c")

        def searchsorted(track_v, T_src, n_steps, base_row,
                         idxL_v, idxR_v, w_v):
            def chunk(i, carry):
                t16 = tgt_v[pl.ds(i * 16, 16)]
                lo = jnp.zeros((16,), jnp.int32)
                hi = jnp.full((16,), T_src, jnp.int32)
                for _ in range(n_steps):
                    mid = jnp.minimum(lax.shift_right_logical(lo + hi, 1),
                                      T_src - 1)
                    tm = plsc.load_gather(track_v, [mid])
                    pred = tm < t16
                    lo = jnp.where(pred, mid + 1, lo)
                    hi = jnp.where(pred, hi, mid)
                idx = jnp.minimum(lo, T_src - 2)
                tl = plsc.load_gather(track_v, [idx])
                tr = plsc.load_gather(track_v, [idx + 1])
                w = jnp.clip((t16 - tl) / (tr - tl + 1e-8), 0.0, 1.0)
                idxL_v[pl.ds(i * 16, 16)] = idx + base_row
                if idxR_v is not None:
                    idxR_v[pl.ds(i * 16, 16)] = idx + 1 + base_row
                w_v[pl.ds(i * 16, 16)] = w
                return carry
            lax.fori_loop(0, T // 16, chunk, 0)

        def sample(s, carry):
            b = wid * SPW + s
            pltpu.sync_copy(tgt.at[pl.ds(b * T, T)], tgt_v)
            pltpu.sync_copy(vis_t.at[pl.ds(b * T_vis, T_vis)], vist_v)
            pltpu.sync_copy(prop_t.at[pl.ds(b * T_prop, T_prop)], propt_v)
            pltpu.sync_copy(
                prop_f.at[pl.ds(b * T_prop * D_prop, T_prop * D_prop)],
                propflat_v)

            searchsorted(vist_v, T_vis, 8, b * T_vis,
                         visidxL_v, visidxR_v, visw_v)
            searchsorted(propt_v, T_prop, 10, 0,
                         propidx_v, None, propw_v)

            def zero(c, carry):
                langsum_v[pl.ds(c * 16, 16)] = jnp.zeros((16,), jnp.float32)
                return carry
            lax.fori_loop(0, D_lang // 16, zero, 0)
            row0 = 0
            while row0 < L:
                rows = min(LCH, L - row0)
                pltpu.sync_copy(
                    lang.at[pl.ds((b * L + row0) * D_lang, rows * D_lang)],
                    langstage_v.at[pl.ds(0, rows * D_lang)])

                def acc(r, carry):
                    for c in range(D_lang // 16):
                        sl = pl.ds(c * 16, 16)
                        langsum_v[sl] = (
                            langsum_v[sl]
                            + langstage_v[pl.ds(r * D_lang + c * 16, 16)])
                    return carry
                lax.fori_loop(0, rows, acc, 0)
                row0 += rows

            def scale(c, carry):
                sl = pl.ds(c * 16, 16)
                langsum_v[sl] = langsum_v[sl] * inv_L
                return carry
            lax.fori_loop(0, D_lang // 16, scale, 0)

            def fill(r, carry):
                for c in range(D_lang // 16):
                    fused_v[r, pl.ds(D_vis + D_prop + c * 16, 16)] = (
                        langsum_v[pl.ds(c * 16, 16)])
                return carry
            lax.fori_loop(0, CH, fill, 0)

            lane = lax.iota(jnp.int32, 16)
            for k in range(NCH):
                cvL = pltpu.async_copy(
                    vis_f.at[visidxL_v.at[pl.ds(k * CH, CH)]], visL_v, sem)
                cvR = pltpu.async_copy(
                    vis_f.at[visidxR_v.at[pl.ds(k * CH, CH)]], visR_v, sem)
                cvL.wait()
                cvR.wait()

                def lerp(r, carry):
                    g = _splat16(k * CH + r)
                    wv = plsc.load_gather(visw_v, [g])
                    for c in range(D_vis // 16):
                        sl = pl.ds(c * 16, 16)
                        lv = visL_v[r, sl]
                        rv = visR_v[r, sl]
                        fused_v[r, sl] = lv + wv * (rv - lv)
                    wp = plsc.load_gather(propw_v, [g])
                    pidx = plsc.load_gather(propidx_v, [g])
                    base = pidx * D_prop + lane
                    for c in range(D_prop // 16):
                        lv = plsc.load_gather(propflat_v, [base + c * 16])
                        rv = plsc.load_gather(propflat_v,
                                              [base + c * 16 + D_prop])
                        fused_v[r, pl.ds(D_vis + c * 16, 16)] = (
                            lv + wp * (rv - lv))
                    return carry
                lax.fori_loop(0, CH, lerp, 0)

                pltpu.sync_copy(fused_v,
                                out.at[pl.ds(b * T + k * CH, CH), :])
            return carry

        lax.fori_loop(0, SPW, sample, 0)

    mesh = plsc.VectorSubcoreMesh(core_axis_name="c", subcore_axis_name="s")
    return pl.kernel(
        body,
        out_type=jax.ShapeDtypeStruct((B * T, D_out), jnp.float32),
        mesh=mesh,
        compiler_params=pltpu.CompilerParams(needs_layout_passes=False),
        scratch_types=[
            pltpu.VMEM((T,), jnp.float32),
            pltpu.VMEM((T_vis,), jnp.float32),
            pltpu.VMEM((T_prop,), jnp.float32),
            pltpu.VMEM((T,), jnp.int32),
            pltpu.VMEM((T,), jnp.int32),
            pltpu.VMEM((T,), jnp.float32),
            pltpu.VMEM((T,), jnp.int32),
            pltpu.VMEM((T,), jnp.float32),
            pltpu.VMEM((T_prop * D_prop,), jnp.float32),
            pltpu.VMEM((D_lang,), jnp.float32),
            pltpu.VMEM((LCH * D_lang,), jnp.float32),
            pltpu.VMEM((CH, D_out), jnp.float32),
            pltpu.VMEM((CH, D_vis), jnp.float32),
            pltpu.VMEM((CH, D_vis), jnp.float32),
            pltpu.SemaphoreType.DMA,
        ],
    )


def kernel(vision_features, vision_timestamps, proprio_features,
           proprio_timestamps, lang_embeddings, target_timestamps):
    B, T_vis, D_vis = vision_features.shape
    _, T_prop, D_prop = proprio_features.shape
    _, L, D_lang = lang_embeddings.shape
    T = target_timestamps.shape[1]
    D_out = D_vis + D_prop + D_lang

    k = _make_sc_kernel(B, T, T_vis, D_vis, T_prop, D_prop, L, D_lang)
    fused = k(vision_features.reshape(B * T_vis, D_vis),
              vision_timestamps.reshape(B * T_vis),
              proprio_features.reshape(B * T_prop * D_prop),
              proprio_timestamps.reshape(B * T_prop),
              lang_embeddings.reshape(B * L * D_lang),
              target_timestamps.reshape(B * T))
    return fused.reshape(B, T, D_out)
